# Initial kernel scaffold; baseline (speedup 1.0000x reference)
import jax
import jax.numpy as jnp
from jax.experimental import pallas as pl


def kernel(notes, onsets, durations, note_embedding_weight):
    raise NotImplementedError("write your pallas kernel here")


# speedup vs baseline: 2.5919x; 2.5919x over previous
import functools

import jax
import jax.numpy as jnp
from jax import lax
from jax.experimental import pallas as pl
from jax.experimental.pallas import tpu as pltpu
from jax.experimental.pallas import tpu_sc as plsc

NUM_NOTES = 91
OUT_D = 10
LANES = 16
NW = 32
IDX_W = 128


@functools.lru_cache(maxsize=None)
def _build(N):
    n_per_w = N // NW
    C = 5120
    n_chunks = n_per_w // C
    steps = C // IDX_W
    rows_per_w = n_per_w // IDX_W

    mesh = plsc.VectorSubcoreMesh(core_axis_name="c", subcore_axis_name="s")

    @functools.partial(
        pl.kernel,
        mesh=mesh,
        out_type=jax.ShapeDtypeStruct((N, OUT_D), jnp.float32),
        scratch_types=[
            pltpu.VMEM((steps, IDX_W), jnp.int32),
            pltpu.VMEM((C,), jnp.float32),
            pltpu.VMEM((C,), jnp.float32),
            pltpu.VMEM((C, OUT_D), jnp.float32),
            pltpu.SemaphoreType.DMA,
        ],
        compiler_params=pltpu.CompilerParams(
            needs_layout_passes=False, use_tc_tiling_on_sc=False
        ),
    )
    def k(table_hbm, notes_hbm, on_hbm, dur_hbm, out_hbm,
          idx_v, on_v, dur_v, out_v, sem):
        wid = lax.axis_index("s") * 2 + lax.axis_index("c")
        iota = lax.iota(jnp.int32, LANES)
        col8 = jnp.full((LANES,), 8, jnp.int32)
        col9 = jnp.full((LANES,), 9, jnp.int32)

        def chunk_body(g, _):
            base = pl.multiple_of(wid * n_per_w + g * C, C)
            row_base = pl.multiple_of(wid * rows_per_w + g * steps, steps)
            pltpu.sync_copy(notes_hbm.at[pl.ds(row_base, steps)], idx_v)
            pltpu.sync_copy(on_hbm.at[pl.ds(base, C)], on_v)
            pltpu.sync_copy(dur_hbm.at[pl.ds(base, C)], dur_v)
            copies = [
                pltpu.async_copy(
                    table_hbm.at[idx_v.at[j]],
                    out_v.at[pl.ds(j * IDX_W, IDX_W)],
                    sem,
                )
                for j in range(steps)
            ]
            for cp in copies:
                cp.wait()

            def scat_body(i, _):
                off = pl.multiple_of(i * LANES, LANES)
                o = on_v[pl.ds(off, LANES)]
                d = dur_v[pl.ds(off, LANES)]
                rows = iota + i * LANES
                plsc.store_scatter(out_v, [rows, col8], o)
                plsc.store_scatter(out_v, [rows, col9], d)
                return 0

            lax.fori_loop(0, C // LANES, scat_body, 0)
            pltpu.sync_copy(out_v, out_hbm.at[pl.ds(base, C)])
            return 0

        lax.fori_loop(0, n_chunks, chunk_body, 0)

    return k


@jax.jit
def kernel(notes, onsets, durations, note_embedding_weight):
    B, L = notes.shape
    N = B * L
    table10 = jnp.pad(note_embedding_weight, ((0, 0), (0, OUT_D - 8)))
    notes2 = notes.reshape(N // IDX_W, IDX_W)
    on = onsets.reshape(N)
    dur = durations.reshape(N)
    out = _build(N)(table10, notes2, on, dur)
    return out.reshape(B, L, OUT_D)


# speedup vs baseline: 19.8593x; 7.6621x over previous
import functools

import jax
import jax.numpy as jnp
from jax import lax
from jax.experimental import pallas as pl
from jax.experimental.pallas import tpu as pltpu
from jax.experimental.pallas import tpu_sc as plsc

NUM_NOTES = 91
EMB = 8
OUT_D = 10
LANES = 16
NW = 32


@functools.lru_cache(maxsize=None)
def _build(N):
    n_per_w = N // NW
    C = 6400
    n_chunks = n_per_w // C

    mesh = plsc.VectorSubcoreMesh(core_axis_name="c", subcore_axis_name="s")

    @functools.partial(
        pl.kernel,
        mesh=mesh,
        out_type=jax.ShapeDtypeStruct((OUT_D * N,), jnp.float32),
        scratch_types=[
            pltpu.VMEM((NUM_NOTES * EMB,), jnp.float32),
            pltpu.VMEM((C,), jnp.int32),
            [pltpu.VMEM((C,), jnp.float32) for _ in range(OUT_D)],
        ],
        compiler_params=pltpu.CompilerParams(
            needs_layout_passes=False, use_tc_tiling_on_sc=False
        ),
    )
    def k(tab_hbm, notes_hbm, on_hbm, dur_hbm, out_hbm, tab_v, notes_v, p_v):
        wid = lax.axis_index("s") * 2 + lax.axis_index("c")
        pltpu.sync_copy(tab_hbm, tab_v)

        def chunk_body(g, _):
            base = pl.multiple_of(wid * n_per_w + g * C, C)
            pltpu.sync_copy(notes_hbm.at[pl.ds(base, C)], notes_v)
            pltpu.sync_copy(on_hbm.at[pl.ds(base, C)], p_v[8])
            pltpu.sync_copy(dur_hbm.at[pl.ds(base, C)], p_v[9])

            def gat_body(i, _):
                off = pl.multiple_of(i * LANES, LANES)
                nt = notes_v[pl.ds(off, LANES)]
                for d in range(EMB):
                    e = plsc.load_gather(tab_v, [nt + (NUM_NOTES * d)])
                    p_v[d][pl.ds(off, LANES)] = e
                return 0

            lax.fori_loop(0, C // LANES, gat_body, 0)
            for d in range(OUT_D):
                pltpu.sync_copy(p_v[d], out_hbm.at[pl.ds(d * N + base, C)])
            return 0

        lax.fori_loop(0, n_chunks, chunk_body, 0)

    return k


@jax.jit
def kernel(notes, onsets, durations, note_embedding_weight):
    B, L = notes.shape
    N = B * L
    tab_cm = note_embedding_weight.T.reshape(NUM_NOTES * EMB)
    notes_p = notes.T.reshape(N)
    on_p = onsets[:, :, 0].T.reshape(N)
    dur_p = durations[:, :, 0].T.reshape(N)
    out = _build(N)(tab_cm, notes_p, on_p, dur_p)
    return jnp.transpose(out.reshape(OUT_D, L, B), (2, 1, 0))


# speedup vs baseline: 20.1082x; 1.0125x over previous
import functools

import jax
import jax.numpy as jnp
from jax import lax
from jax.experimental import pallas as pl
from jax.experimental.pallas import tpu as pltpu
from jax.experimental.pallas import tpu_sc as plsc

NUM_NOTES = 91
EMB = 8
OUT_D = 10
LANES = 16
NW = 32


@functools.lru_cache(maxsize=None)
def _build(N):
    n_per_w = N // NW
    C = 6400
    n_chunks = n_per_w // C

    mesh = plsc.VectorSubcoreMesh(core_axis_name="c", subcore_axis_name="s")

    @functools.partial(
        pl.kernel,
        mesh=mesh,
        out_type=jax.ShapeDtypeStruct((OUT_D, N), jnp.float32),
        scratch_types=[
            pltpu.VMEM((NUM_NOTES * EMB,), jnp.float32),
            pltpu.VMEM((C,), jnp.int32),
            pltpu.VMEM((OUT_D, C), jnp.float32),
        ],
        compiler_params=pltpu.CompilerParams(
            needs_layout_passes=False, use_tc_tiling_on_sc=False
        ),
    )
    def k(tab_hbm, notes_hbm, on_hbm, dur_hbm, out_hbm, tab_v, notes_v, p_v):
        wid = lax.axis_index("s") * 2 + lax.axis_index("c")
        pltpu.sync_copy(tab_hbm, tab_v)

        def chunk_body(g, _):
            base = pl.multiple_of(wid * n_per_w + g * C, C)
            pltpu.sync_copy(notes_hbm.at[pl.ds(base, C)], notes_v)
            pltpu.sync_copy(on_hbm.at[pl.ds(base, C)], p_v.at[EMB])
            pltpu.sync_copy(dur_hbm.at[pl.ds(base, C)], p_v.at[EMB + 1])

            def gat_body(i, _):
                off = pl.multiple_of(i * LANES, LANES)
                nt = notes_v[pl.ds(off, LANES)]
                for d in range(EMB):
                    e = plsc.load_gather(tab_v, [nt + (NUM_NOTES * d)])
                    p_v[d, pl.ds(off, LANES)] = e
                return 0

            lax.fori_loop(0, C // LANES, gat_body, 0)
            pltpu.sync_copy(p_v, out_hbm.at[:, pl.ds(base, C)])
            return 0

        lax.fori_loop(0, n_chunks, chunk_body, 0)

    return k


@jax.jit
def kernel(notes, onsets, durations, note_embedding_weight):
    B, L = notes.shape
    N = B * L
    tab_cm = note_embedding_weight.T.reshape(NUM_NOTES * EMB)
    notes_p = notes.T.reshape(N)
    on_p = onsets[:, :, 0].T.reshape(N)
    dur_p = durations[:, :, 0].T.reshape(N)
    out = _build(N)(tab_cm, notes_p, on_p, dur_p)
    return jnp.transpose(out.reshape(OUT_D, L, B), (2, 1, 0))


# speedup vs baseline: 27.3937x; 1.3623x over previous
import functools

import jax
import jax.numpy as jnp
from jax import lax
from jax.experimental import pallas as pl
from jax.experimental.pallas import tpu as pltpu
from jax.experimental.pallas import tpu_sc as plsc

NUM_NOTES = 91
EMB = 8
OUT_D = 10
LANES = 16
NW = 32


@functools.lru_cache(maxsize=None)
def _build(N):
    n_per_w = N // NW
    C = 6400
    n_chunks = n_per_w // C

    mesh = plsc.VectorSubcoreMesh(core_axis_name="c", subcore_axis_name="s")

    @functools.partial(
        pl.kernel,
        mesh=mesh,
        out_type=jax.ShapeDtypeStruct((OUT_D, N), jnp.float32),
        scratch_types=[
            pltpu.VMEM((NUM_NOTES * EMB,), jnp.float32),
            pltpu.VMEM((C,), jnp.int32),
            pltpu.VMEM((OUT_D, C), jnp.float32),
        ],
        compiler_params=pltpu.CompilerParams(
            needs_layout_passes=False, use_tc_tiling_on_sc=False
        ),
    )
    def k(tab_hbm, notes_hbm, on_hbm, dur_hbm, out_hbm, tab_v, notes_v, p_v):
        wid = lax.axis_index("s") * 2 + lax.axis_index("c")
        pltpu.sync_copy(tab_hbm, tab_v)

        def chunk_body(g, _):
            base = pl.multiple_of(wid * n_per_w + g * C, C)
            pltpu.sync_copy(notes_hbm.at[pl.ds(base, C)], notes_v)
            pltpu.sync_copy(on_hbm.at[pl.ds(base, C)], p_v.at[EMB])
            pltpu.sync_copy(dur_hbm.at[pl.ds(base, C)], p_v.at[EMB + 1])

            def gat_body(i, _):
                off = pl.multiple_of(i * LANES, LANES)
                nt = notes_v[pl.ds(off, LANES)]
                for d in range(EMB):
                    e = plsc.load_gather(tab_v, [nt + (NUM_NOTES * d)])
                    p_v[d, pl.ds(off, LANES)] = e
                return 0

            lax.fori_loop(0, C // LANES, gat_body, 0)
            pltpu.sync_copy(p_v, out_hbm.at[:, pl.ds(base, C)])
            return 0

        lax.fori_loop(0, n_chunks, chunk_body, 0)

    return k


def _tile_order(x, L, B):
    return x.T.reshape(L // 8, 8, B // 128, 128).transpose(0, 2, 1, 3).reshape(L * B)


@jax.jit
def kernel(notes, onsets, durations, note_embedding_weight):
    B, L = notes.shape
    N = B * L
    tab_cm = note_embedding_weight.T.reshape(NUM_NOTES * EMB)
    notes_p = _tile_order(notes, L, B)
    on_p = _tile_order(onsets[:, :, 0], L, B)
    dur_p = _tile_order(durations[:, :, 0], L, B)
    out = _build(N)(tab_cm, notes_p, on_p, dur_p)
    out5 = out.reshape(OUT_D, L // 8, B // 128, 8, 128)
    return out5.transpose(2, 4, 1, 3, 0).reshape(B, L, OUT_D)
